# baseline (speedup 1.0000x reference)
import functools

import jax
import jax.numpy as jnp
from jax import lax
from jax.experimental import pallas as pl
from jax.experimental.pallas import tpu as pltpu
from jax.experimental.pallas import tpu_sc as plsc

D = 128
DEG = 32
G = 3
GW = G * DEG
NC = 2
NS = 16
NW = NC * NS
NPAD = 10112
STG = NPAD // NS


def _mm_body(x_ref, w_ref, b_ref, o_ref):
    o_ref[...] = (
        jnp.dot(x_ref[...], w_ref[0], preferred_element_type=jnp.float32)
        + b_ref[0]
    )


def _project_stacked(feat_pad, wstack, bstack, nb):
    bm = NPAD // nb
    return pl.pallas_call(
        _mm_body,
        grid=(2, nb),
        in_specs=[
            pl.BlockSpec((bm, D), lambda g, i: (i, 0)),
            pl.BlockSpec((1, D, D), lambda g, i: (g, 0, 0)),
            pl.BlockSpec((1, 1, D), lambda g, i: (g, 0, 0)),
        ],
        out_specs=pl.BlockSpec((bm, D), lambda g, i: (g * nb + i, 0)),
        out_shape=jax.ShapeDtypeStruct((2 * NPAD, D), jnp.float32),
    )(feat_pad, wstack, bstack)


def _bag_sum(self_ref, slab, g, stage, out_v):
    for cc in range(D // 16):
        sl = pl.ds(cc * 16, 16)
        vals = [self_ref[stage + g, sl]]
        vals += [slab[DEG * g + j, sl] for j in range(DEG)]
        while len(vals) > 1:
            nxt = [vals[i] + vals[i + 1] for i in range(0, len(vals) - 1, 2)]
            if len(vals) % 2:
                nxt.append(vals[-1])
            vals = nxt
        out_v[stage + g, sl] = jnp.maximum(vals[0], 0.0)


def _make_bag_kernel(ni, b_per_w, bpad):
    mesh = plsc.VectorSubcoreMesh(core_axis_name="c", subcore_axis_name="s")
    CH = 8 * G
    nch = ni // 8

    @functools.partial(
        pl.kernel,
        mesh=mesh,
        out_type=jax.ShapeDtypeStruct((bpad, D), jnp.float32),
        scratch_types=[
            pltpu.VMEM_SHARED((NPAD, D), jnp.float32),
            pltpu.VMEM((ni, GW), jnp.int32),
            pltpu.VMEM((b_per_w,), jnp.int32),
            pltpu.VMEM((2, CH, D), jnp.float32),
            pltpu.VMEM((2, GW, D), jnp.float32),
            pltpu.VMEM((CH, D), jnp.float32),
            pltpu.SemaphoreType.DMA,
            pltpu.SemaphoreType.DMA,
            pltpu.SemaphoreType.DMA,
            pltpu.SemaphoreType.DMA,
        ],
    )
    def bag(
        t_hbm, nidx_hbm, sidx_hbm, out_hbm,
        tn_sp, nidx_v, sidx_v, self_v, rows_v, out_v, sem0, sem1, ss0, ss1,
    ):
        cid = lax.axis_index("c")
        sid = lax.axis_index("s")
        wid = sid * NC + cid

        pltpu.sync_copy(sidx_hbm.at[wid], sidx_v)
        pltpu.async_copy(
            t_hbm.at[sidx_v.at[pl.ds(0, CH)]], self_v.at[0], ss0
        )
        pltpu.async_copy(
            t_hbm.at[sidx_v.at[pl.ds(CH, CH)]], self_v.at[1], ss1
        )
        pltpu.sync_copy(nidx_hbm.at[wid], nidx_v)
        pltpu.sync_copy(
            t_hbm.at[pl.ds(NPAD + sid * STG, STG)],
            tn_sp.at[pl.ds(sid * STG, STG)],
        )
        plsc.subcore_barrier()

        HW = GW // 2

        def nfire(t, buf, sem):
            pltpu.async_copy(
                tn_sp.at[nidx_v.at[t, pl.ds(0, HW)]],
                rows_v.at[buf, pl.ds(0, HW)], sem,
            )
            pltpu.async_copy(
                tn_sp.at[nidx_v.at[t, pl.ds(HW, HW)]],
                rows_v.at[buf, pl.ds(HW, HW)], sem,
            )

        def ndrain(t, buf, sem):
            pltpu.make_async_copy(
                tn_sp.at[nidx_v.at[t, pl.ds(0, HW)]],
                rows_v.at[buf, pl.ds(0, HW)], sem,
            ).wait()
            pltpu.make_async_copy(
                tn_sp.at[nidx_v.at[t, pl.ds(HW, HW)]],
                rows_v.at[buf, pl.ds(HW, HW)], sem,
            ).wait()

        nfire(0, 0, sem0)
        nfire(1, 1, sem1)

        def self_wait(m, sbuf, sem):
            pltpu.make_async_copy(
                t_hbm.at[sidx_v.at[pl.ds(m * CH, CH)]],
                self_v.at[sbuf],
                sem,
            ).wait()

        def self_fire(m, sbuf, sem):
            @pl.when(m < nch)
            def _():
                pltpu.async_copy(
                    t_hbm.at[sidx_v.at[pl.ds(m * CH, CH)]],
                    self_v.at[sbuf],
                    sem,
                )

        sems = (sem0, sem1)

        def body(pp, carry):
            sbuf = pp % 2

            @pl.when(sbuf == 0)
            def _():
                self_wait(pp, 0, ss0)

            @pl.when(sbuf == 1)
            def _():
                self_wait(pp, 1, ss1)

            for q in range(8):
                t = 8 * pp + q
                buf = q % 2
                ndrain(t, buf, sems[buf])
                slab = rows_v.at[buf]
                for g in range(G):
                    _bag_sum(self_v.at[sbuf], slab, g, q * G, out_v)

                @pl.when(t + 2 < ni)
                def _():
                    nfire(t + 2, buf, sems[buf])

            pltpu.sync_copy(
                out_v,
                out_hbm.at[pl.ds(wid * b_per_w + pp * CH, CH)],
            )

            @pl.when(sbuf == 0)
            def _():
                self_fire(pp + 2, 0, ss0)

            @pl.when(sbuf == 1)
            def _():
                self_fire(pp + 2, 1, ss1)

            return carry

        lax.fori_loop(0, nch, body, 0)

    return bag


def kernel(feat_table, W1, b1, nodes, neigh_index):
    n_nodes, d = feat_table.shape
    b = nodes.shape[0]
    ni = -(-b // (NW * G * 8)) * 8
    b_per_w = ni * G
    bpad = NW * b_per_w

    wt = W1.T.astype(jnp.float32)
    feat_pad = jnp.concatenate(
        [feat_table, jnp.zeros((NPAD - n_nodes, d), jnp.float32)]
    )
    wstack = jnp.stack([wt[:d], wt[d:] * (1.0 / DEG)])[:, None]
    wstack = wstack.reshape(2, d, d)
    bstack = jnp.stack([b1, jnp.zeros_like(b1)])[:, None, :]
    tstk = _project_stacked(feat_pad, wstack, bstack, 8)

    nrow_pad = bpad - b
    sidx = jnp.concatenate(
        [
            nodes.astype(jnp.int32),
            jnp.arange(nrow_pad, dtype=jnp.int32) % jnp.int32(n_nodes),
        ]
    ).reshape(NW, b_per_w)
    nfill = (
        jnp.arange(nrow_pad * DEG, dtype=jnp.int32) % jnp.int32(n_nodes)
    ).reshape(nrow_pad, DEG)
    nidx = jnp.concatenate([neigh_index.astype(jnp.int32), nfill], axis=0)
    nidx = nidx.reshape(NW, ni, GW)

    out = _make_bag_kernel(ni, b_per_w, bpad)(tstk, nidx, sidx)
    return out[:b]


# speedup vs baseline: 1.0154x; 1.0154x over previous
import functools

import jax
import jax.numpy as jnp
from jax import lax
from jax.experimental import pallas as pl
from jax.experimental.pallas import tpu as pltpu
from jax.experimental.pallas import tpu_sc as plsc

D = 128
DEG = 32
G = 3
GW = G * DEG
NC = 2
NS = 16
NW = NC * NS
NPAD = 10112
STG = NPAD // NS


def _mm_body(x_ref, w_ref, b_ref, o_ref):
    o_ref[...] = (
        jnp.dot(x_ref[...], w_ref[0], preferred_element_type=jnp.float32)
        + b_ref[0]
    )


def _project_stacked(feat_pad, wstack, bstack, nb):
    bm = NPAD // nb
    return pl.pallas_call(
        _mm_body,
        grid=(2, nb),
        in_specs=[
            pl.BlockSpec((bm, D), lambda g, i: (i, 0)),
            pl.BlockSpec((1, D, D), lambda g, i: (g, 0, 0)),
            pl.BlockSpec((1, 1, D), lambda g, i: (g, 0, 0)),
        ],
        out_specs=pl.BlockSpec((bm, D), lambda g, i: (g * nb + i, 0)),
        out_shape=jax.ShapeDtypeStruct((2 * NPAD, D), jnp.float32),
    )(feat_pad, wstack, bstack)


def _bag_sum(self_ref, slab, g, stage, out_v):
    for cc in range(D // 16):
        sl = pl.ds(cc * 16, 16)
        vals = [self_ref[stage + g, sl]]
        vals += [slab[DEG * g + j, sl] for j in range(DEG)]
        while len(vals) > 1:
            nxt = [vals[i] + vals[i + 1] for i in range(0, len(vals) - 1, 2)]
            if len(vals) % 2:
                nxt.append(vals[-1])
            vals = nxt
        out_v[stage + g, sl] = jnp.maximum(vals[0], 0.0)


def _make_bag_kernel(ni, b_per_w, bpad):
    mesh = plsc.VectorSubcoreMesh(core_axis_name="c", subcore_axis_name="s")
    CH = 8 * G
    nch = ni // 8

    @functools.partial(
        pl.kernel,
        mesh=mesh,
        out_type=jax.ShapeDtypeStruct((bpad, D), jnp.float32),
        scratch_types=[
            pltpu.VMEM_SHARED((NPAD, D), jnp.float32),
            pltpu.VMEM((ni, GW), jnp.int32),
            pltpu.VMEM((b_per_w,), jnp.int32),
            pltpu.VMEM((2, CH, D), jnp.float32),
            pltpu.VMEM((2, GW, D), jnp.float32),
            pltpu.VMEM((CH, D), jnp.float32),
            pltpu.SemaphoreType.DMA,
            pltpu.SemaphoreType.DMA,
            pltpu.SemaphoreType.DMA,
            pltpu.SemaphoreType.DMA,
        ],
    )
    def bag(
        t_hbm, nidx_hbm, sidx_hbm, out_hbm,
        tn_sp, nidx_v, sidx_v, self_v, rows_v, out_v, sem0, sem1, ss0, ss1,
    ):
        cid = lax.axis_index("c")
        sid = lax.axis_index("s")
        wid = sid * NC + cid

        pltpu.sync_copy(sidx_hbm.at[wid], sidx_v)
        pltpu.async_copy(
            t_hbm.at[sidx_v.at[pl.ds(0, CH)]], self_v.at[0], ss0
        )
        pltpu.async_copy(
            t_hbm.at[sidx_v.at[pl.ds(CH, CH)]], self_v.at[1], ss1
        )
        pltpu.sync_copy(nidx_hbm.at[wid], nidx_v)
        pltpu.sync_copy(
            t_hbm.at[pl.ds(NPAD + sid * STG, STG)],
            tn_sp.at[pl.ds(sid * STG, STG)],
        )
        plsc.subcore_barrier()

        HW = GW // 2

        def nfire(t, buf, sem):
            pltpu.async_copy(
                tn_sp.at[nidx_v.at[t, pl.ds(0, HW)]],
                rows_v.at[buf, pl.ds(0, HW)], sem,
            )
            pltpu.async_copy(
                tn_sp.at[nidx_v.at[t, pl.ds(HW, HW)]],
                rows_v.at[buf, pl.ds(HW, HW)], sem,
            )

        def ndrain(t, buf, sem):
            pltpu.make_async_copy(
                tn_sp.at[nidx_v.at[t, pl.ds(0, HW)]],
                rows_v.at[buf, pl.ds(0, HW)], sem,
            ).wait()
            pltpu.make_async_copy(
                tn_sp.at[nidx_v.at[t, pl.ds(HW, HW)]],
                rows_v.at[buf, pl.ds(HW, HW)], sem,
            ).wait()

        nfire(0, 0, sem0)
        nfire(1, 1, sem1)

        def self_wait(m, sbuf, sem):
            pltpu.make_async_copy(
                t_hbm.at[sidx_v.at[pl.ds(m * CH, CH)]],
                self_v.at[sbuf],
                sem,
            ).wait()

        def self_fire(m, sbuf, sem):
            @pl.when(m < nch)
            def _():
                pltpu.async_copy(
                    t_hbm.at[sidx_v.at[pl.ds(m * CH, CH)]],
                    self_v.at[sbuf],
                    sem,
                )

        sems = (sem0, sem1)

        def body(pp, carry):
            sbuf = pp % 2

            @pl.when(sbuf == 0)
            def _():
                self_wait(pp, 0, ss0)

            @pl.when(sbuf == 1)
            def _():
                self_wait(pp, 1, ss1)

            for q in range(8):
                t = 8 * pp + q
                buf = q % 2
                ndrain(t, buf, sems[buf])
                slab = rows_v.at[buf]
                for g in range(G):
                    _bag_sum(self_v.at[sbuf], slab, g, q * G, out_v)

                @pl.when(t + 2 < ni)
                def _():
                    nfire(t + 2, buf, sems[buf])

            pltpu.sync_copy(
                out_v,
                out_hbm.at[pl.ds(wid * b_per_w + pp * CH, CH)],
            )

            @pl.when(sbuf == 0)
            def _():
                self_fire(pp + 2, 0, ss0)

            @pl.when(sbuf == 1)
            def _():
                self_fire(pp + 2, 1, ss1)

            return carry

        lax.fori_loop(0, nch, body, 0)

    return bag


def kernel(feat_table, W1, b1, nodes, neigh_index):
    n_nodes, d = feat_table.shape
    b = nodes.shape[0]
    ni = -(-b // (NW * G * 8)) * 8
    b_per_w = ni * G
    bpad = NW * b_per_w

    wt = W1.T.astype(jnp.float32)
    feat_pad = jnp.concatenate(
        [feat_table, jnp.zeros((NPAD - n_nodes, d), jnp.float32)]
    )
    wstack = jnp.stack([wt[:d], wt[d:] * (1.0 / DEG)])[:, None]
    wstack = wstack.reshape(2, d, d)
    bstack = jnp.stack([b1, jnp.zeros_like(b1)])[:, None, :]
    tstk = _project_stacked(
        feat_pad.astype(jnp.bfloat16), wstack.astype(jnp.bfloat16), bstack, 8
    )

    nrow_pad = bpad - b
    sidx = jnp.concatenate(
        [
            nodes.astype(jnp.int32),
            jnp.arange(nrow_pad, dtype=jnp.int32) % jnp.int32(n_nodes),
        ]
    ).reshape(NW, b_per_w)
    nfill = (
        jnp.arange(nrow_pad * DEG, dtype=jnp.int32) % jnp.int32(n_nodes)
    ).reshape(nrow_pad, DEG)
    nidx = jnp.concatenate([neigh_index.astype(jnp.int32), nfill], axis=0)
    nidx = nidx.reshape(NW, ni, GW)

    out = _make_bag_kernel(ni, b_per_w, bpad)(tstk, nidx, sidx)
    return out[:b]
